# Initial kernel scaffold; baseline (speedup 1.0000x reference)
import jax
import jax.numpy as jnp
from jax.experimental import pallas as pl
from jax.experimental.pallas import tpu as pltpu
from jax.experimental.pallas import tpu_sc as plsc

_BATCH = 16384
_FIELDS = 26
_DIM = 32
_N = _BATCH * _FIELDS
_WINDOW = 512


def _sc_gather(dictionary, idx_flat):
    mesh = plsc.VectorSubcoreMesh(core_axis_name="core", subcore_axis_name="subcore")

    @pl.kernel(
        out_type=jax.ShapeDtypeStruct((_N, _DIM), dictionary.dtype),
        mesh=mesh,
    )
    def gather_kernel(table_hbm, idx_hbm, out_hbm):
        def body(idx_vmem, out_vmem):
            pltpu.sync_copy(table_hbm.at[idx_vmem.at[0]], out_vmem)

        pltpu.emit_pipeline(
            body,
            grid=(_N // _WINDOW,),
            in_specs=[pl.BlockSpec((1, _WINDOW), index_map=lambda i: (0, i))],
            out_specs=[pl.BlockSpec((_WINDOW, _DIM), index_map=lambda i: (i, 0))],
            core_axis_name=("core", "subcore"),
            dimension_semantics=(pltpu.PARALLEL,),
        )(idx_hbm, out_hbm)

    return gather_kernel(dictionary, idx_flat)


def kernel(x, dictionary):
    idx_flat = x.astype(jnp.int32).reshape(1, _N)
    out = _sc_gather(dictionary, idx_flat)
    return out.reshape(_BATCH, _FIELDS, _DIM)


# speedup vs baseline: 1.5484x; 1.5484x over previous
import jax
import jax.numpy as jnp
from jax import lax
from jax.experimental import pallas as pl
from jax.experimental.pallas import tpu as pltpu
from jax.experimental.pallas import tpu_sc as plsc

_BATCH = 16384
_FIELDS = 26
_DIM = 32
_N = _BATCH * _FIELDS
_NC = 2
_NS = 16
_NW = _NC * _NS
_B_PER_W = _N // _NW
_CHUNK = 1024
_N_CHUNKS = _B_PER_W // _CHUNK


def _sc_gather(dictionary, idx_flat):
    mesh = plsc.VectorSubcoreMesh(core_axis_name="c", subcore_axis_name="s")

    @pl.kernel(
        out_type=jax.ShapeDtypeStruct((_N, _DIM), dictionary.dtype),
        mesh=mesh,
        scratch_types=[
            pltpu.VMEM((_CHUNK,), jnp.int32),
            pltpu.VMEM((_CHUNK, _DIM), jnp.float32),
            pltpu.SemaphoreType.DMA,
        ],
        compiler_params=pltpu.CompilerParams(use_tc_tiling_on_sc=False),
    )
    def gather_kernel(table_hbm, idx_hbm, out_hbm, idx_v, rows_v, sem):
        wid = lax.axis_index("s") * _NC + lax.axis_index("c")
        base = wid * _B_PER_W

        @pl.loop(0, _N_CHUNKS)
        def _(j):
            off = base + j * _CHUNK
            pltpu.sync_copy(idx_hbm.at[pl.ds(off, _CHUNK)], idx_v)
            pltpu.async_copy(table_hbm.at[idx_v], rows_v, sem).wait()
            pltpu.sync_copy(rows_v, out_hbm.at[pl.ds(off, _CHUNK)])

    return gather_kernel(dictionary, idx_flat)


def kernel(x, dictionary):
    idx_flat = x.astype(jnp.int32).reshape(_N)
    out = _sc_gather(dictionary, idx_flat)
    return out.reshape(_BATCH, _FIELDS, _DIM)


# speedup vs baseline: 1.5787x; 1.0196x over previous
import jax
import jax.numpy as jnp
from jax import lax
from jax.experimental import pallas as pl
from jax.experimental.pallas import tpu as pltpu
from jax.experimental.pallas import tpu_sc as plsc

_BATCH = 16384
_FIELDS = 26
_DIM = 32
_N = _BATCH * _FIELDS
_NC = 2
_NS = 16
_NW = _NC * _NS
_B_PER_W = _N // _NW
_CHUNK = 1664
_N_CHUNKS = _B_PER_W // _CHUNK


def _sc_gather(dictionary, idx_flat):
    mesh = plsc.VectorSubcoreMesh(core_axis_name="c", subcore_axis_name="s")

    @pl.kernel(
        out_type=jax.ShapeDtypeStruct((_N, _DIM), dictionary.dtype),
        mesh=mesh,
        scratch_types=[
            pltpu.VMEM((_B_PER_W,), jnp.int32),
            pltpu.VMEM((2, _CHUNK, _DIM), jnp.float32),
            pltpu.SemaphoreType.DMA,
            pltpu.SemaphoreType.DMA,
            pltpu.SemaphoreType.DMA,
            pltpu.SemaphoreType.DMA,
        ],
        compiler_params=pltpu.CompilerParams(use_tc_tiling_on_sc=False),
    )
    def gather_kernel(table_hbm, idx_hbm, out_hbm, idx_v, rows_v,
                      gsem0, gsem1, osem0, osem1):
        wid = lax.axis_index("s") * _NC + lax.axis_index("c")
        base = wid * _B_PER_W
        gsem = (gsem0, gsem1)
        osem = (osem0, osem1)

        pltpu.sync_copy(idx_hbm.at[pl.ds(base, _B_PER_W)], idx_v)

        gathers = {}
        writes = {}
        gathers[0] = pltpu.async_copy(
            table_hbm.at[idx_v.at[pl.ds(0, _CHUNK)]], rows_v.at[0], gsem[0])
        for j in range(_N_CHUNKS):
            b = j % 2
            bn = (j + 1) % 2
            if j + 1 < _N_CHUNKS:
                if j >= 1:
                    writes[j - 1].wait()
                gathers[j + 1] = pltpu.async_copy(
                    table_hbm.at[idx_v.at[pl.ds((j + 1) * _CHUNK, _CHUNK)]],
                    rows_v.at[bn], gsem[bn])
            gathers[j].wait()
            writes[j] = pltpu.async_copy(
                rows_v.at[b], out_hbm.at[pl.ds(base + j * _CHUNK, _CHUNK)],
                osem[b])
        writes[_N_CHUNKS - 2].wait()
        writes[_N_CHUNKS - 1].wait()

    return gather_kernel(dictionary, idx_flat)


def kernel(x, dictionary):
    idx_flat = x.astype(jnp.int32).reshape(_N)
    out = _sc_gather(dictionary, idx_flat)
    return out.reshape(_BATCH, _FIELDS, _DIM)
